# Initial kernel scaffold; baseline (speedup 1.0000x reference)
import jax
import jax.numpy as jnp
from jax.experimental import pallas as pl


def kernel(feat, topk_values, topk_indices, edge_index, W_neigh, W_self, b_self):
    raise NotImplementedError("write your pallas kernel here")


# speedup vs baseline: 12.1203x; 12.1203x over previous
import dataclasses
import functools

import jax
import jax.numpy as jnp
from jax import lax
from jax.experimental import pallas as pl
from jax.experimental.pallas import tpu as pltpu
from jax.experimental.pallas import tpu_sc as plsc

N = 10000
E = 320000
D = 128
K = 32
HALF = 64
L = 16
NSUB = 16
ROW_BLK = 200
N_ROW_BLKS = N // ROW_BLK
ROW_ITERS = (N_ROW_BLKS + NSUB - 1) // NSUB
EDGE_BLK = 128
N_EDGE_BLKS = E // EDGE_BLK
EDGE_ITERS = (N_EDGE_BLKS + NSUB - 1) // NSUB

_mesh = plsc.VectorSubcoreMesh(core_axis_name="c", subcore_axis_name="s")

_sc_params = pltpu.CompilerParams()
if "needs_layout_passes" in pltpu.CompilerParams.__dataclass_fields__:
    _sc_params = dataclasses.replace(_sc_params, needs_layout_passes=False)
if "use_tc_tiling_on_sc" in pltpu.CompilerParams.__dataclass_fields__:
    _sc_params = dataclasses.replace(_sc_params, use_tc_tiling_on_sc=False)


@functools.partial(
    pl.kernel,
    out_type=(
        jax.ShapeDtypeStruct((2, N, HALF), jnp.float32),
        jax.ShapeDtypeStruct((N,), jnp.float32),
    ),
    mesh=_mesh,
    compiler_params=_sc_params,
    scratch_types=[
        pltpu.VMEM_SHARED((N, HALF), jnp.float32),
        pltpu.VMEM_SHARED((N, HALF), jnp.float32),
        pltpu.VMEM_SHARED((N,), jnp.float32),
        pltpu.VMEM((ROW_BLK, HALF), jnp.float32),
        pltpu.VMEM((ROW_BLK, K), jnp.int32),
        pltpu.VMEM((ROW_BLK, K), jnp.float32),
        pltpu.VMEM((EDGE_BLK,), jnp.int32),
        pltpu.VMEM((EDGE_BLK,), jnp.int32),
        pltpu.VMEM((EDGE_BLK, HALF), jnp.float32),
        pltpu.VMEM((EDGE_BLK,), jnp.float32),
        pltpu.VMEM((N // 5,), jnp.float32),
    ],
)
def _sc_aggregate(tv_hbm, ti_hbm, src_hbm, dst_hbm, hs_out, deg_out,
                  sf_sh, hacc_sh, deg_sh,
                  buf, ti_vm, tv_vm, sidx, didx, stage, ones, zdeg):
    c = lax.axis_index("c")
    s = lax.axis_index("s")
    zvec = jnp.zeros((L,), jnp.float32)

    @pl.loop(0, ROW_BLK)
    def _(r):
        for h in range(HALF // L):
            buf[r, pl.ds(h * L, L)] = zvec

    @pl.loop(0, ROW_ITERS)
    def _(i):
        j = i * NSUB + s

        @pl.when(j < N_ROW_BLKS)
        def _():
            pltpu.sync_copy(buf, hacc_sh.at[pl.ds(j * ROW_BLK, ROW_BLK)])

    @pl.when(jnp.logical_and(c == 0, s == 0))
    def _():
        @pl.loop(0, N // 5 // L)
        def _(i):
            zdeg[pl.ds(i * L, L)] = zvec

        @pl.loop(0, 5)
        def _(i):
            pltpu.sync_copy(zdeg, deg_sh.at[pl.ds(i * (N // 5), N // 5)])

    @pl.when(c == 0)
    def _():
        @pl.loop(0, EDGE_BLK // L)
        def _(i):
            ones[pl.ds(i * L, L)] = jnp.ones((L,), jnp.float32)

    coff = c * HALF

    @pl.loop(0, ROW_ITERS)
    def _(i):
        j = i * NSUB + s

        @pl.when(j < N_ROW_BLKS)
        def _():
            base = j * ROW_BLK
            pltpu.sync_copy(ti_hbm.at[pl.ds(base, ROW_BLK)], ti_vm)
            pltpu.sync_copy(tv_hbm.at[pl.ds(base, ROW_BLK)], tv_vm)

            @pl.loop(0, ROW_BLK)
            def _(r):
                row_ids = jnp.full((L,), r, jnp.int32)
                for h in range(K // L):
                    cols = ti_vm[r, pl.ds(h * L, L)] - coff
                    vals = tv_vm[r, pl.ds(h * L, L)]
                    mask = jnp.logical_and(cols >= 0, cols < HALF)
                    plsc.store_scatter(buf, [row_ids, cols], vals, mask=mask)

            pltpu.sync_copy(buf, sf_sh.at[pl.ds(base, ROW_BLK)])

            @pl.loop(0, ROW_BLK)
            def _(r):
                row_ids = jnp.full((L,), r, jnp.int32)
                for h in range(K // L):
                    cols = ti_vm[r, pl.ds(h * L, L)] - coff
                    mask = jnp.logical_and(cols >= 0, cols < HALF)
                    plsc.store_scatter(buf, [row_ids, cols], zvec, mask=mask)

    plsc.subcore_barrier()

    @pl.loop(0, EDGE_ITERS)
    def _(i):
        j = i * NSUB + s

        @pl.when(j < N_EDGE_BLKS)
        def _():
            base = j * EDGE_BLK
            pltpu.sync_copy(src_hbm.at[pl.ds(base, EDGE_BLK)], sidx)
            pltpu.sync_copy(dst_hbm.at[pl.ds(base, EDGE_BLK)], didx)
            pltpu.sync_copy(sf_sh.at[sidx], stage)
            pltpu.sync_copy(stage, hacc_sh.at[didx], add=True)

            @pl.when(c == 0)
            def _():
                pltpu.sync_copy(ones, deg_sh.at[didx], add=True)

    plsc.subcore_barrier()

    @pl.loop(0, ROW_ITERS)
    def _(i):
        j = i * NSUB + s

        @pl.when(j < N_ROW_BLKS)
        def _():
            base = j * ROW_BLK
            pltpu.sync_copy(hacc_sh.at[pl.ds(base, ROW_BLK)],
                            hs_out.at[c].at[pl.ds(base, ROW_BLK)])

    @pl.when(jnp.logical_and(c == 0, s == 0))
    def _():
        pltpu.sync_copy(deg_sh, deg_out)


R_TC = 1000
_TC_GRID = N // R_TC


def _tc_body(feat_ref, h0_ref, h1_ref, w_ref, ws_ref, wn_ref, b_ref, out_ref):
    w = w_ref[...]
    h0w = h0_ref[...] * w
    h1w = h1_ref[...] * w
    dn = (((1,), (1,)), ((), ()))
    acc = lax.dot_general(feat_ref[...], ws_ref[...], dn,
                          preferred_element_type=jnp.float32,
                          precision=lax.Precision.HIGHEST)
    wn = wn_ref[...]
    acc += lax.dot_general(h0w, wn[:, :HALF], dn,
                           preferred_element_type=jnp.float32,
                           precision=lax.Precision.HIGHEST)
    acc += lax.dot_general(h1w, wn[:, HALF:], dn,
                           preferred_element_type=jnp.float32,
                           precision=lax.Precision.HIGHEST)
    out_ref[...] = acc + b_ref[...]


def _tc_combine(feat, h0, h1, winv, W_self, W_neigh, b):
    return pl.pallas_call(
        _tc_body,
        grid=(_TC_GRID,),
        in_specs=[
            pl.BlockSpec((R_TC, D), lambda i: (i, 0)),
            pl.BlockSpec((R_TC, HALF), lambda i: (i, 0)),
            pl.BlockSpec((R_TC, HALF), lambda i: (i, 0)),
            pl.BlockSpec((R_TC, 1), lambda i: (i, 0)),
            pl.BlockSpec((D, D), lambda i: (0, 0)),
            pl.BlockSpec((D, D), lambda i: (0, 0)),
            pl.BlockSpec((1, D), lambda i: (0, 0)),
        ],
        out_specs=pl.BlockSpec((R_TC, D), lambda i: (i, 0)),
        out_shape=jax.ShapeDtypeStruct((N, D), jnp.float32),
    )(feat, h0, h1, winv, W_self, W_neigh, b)


def kernel(feat, topk_values, topk_indices, edge_index, W_neigh, W_self, b_self):
    ti = topk_indices.astype(jnp.int32)
    src = edge_index[0].astype(jnp.int32)
    dst = edge_index[1].astype(jnp.int32)
    hs, deg = _sc_aggregate(topk_values.astype(jnp.float32), ti, src, dst)
    winv = (1.0 / jnp.maximum(deg, 1.0))[:, None]
    return _tc_combine(feat, hs[0], hs[1], winv,
                       W_self, W_neigh, b_self.reshape(1, D))


# speedup vs baseline: 21.5915x; 1.7814x over previous
import dataclasses
import functools

import jax
import jax.numpy as jnp
from jax import lax
from jax.experimental import pallas as pl
from jax.experimental.pallas import tpu as pltpu
from jax.experimental.pallas import tpu_sc as plsc

N = 10000
E = 320000
D = 128
K = 32
HALF = 64
L = 16
NSUB = 16
ROW_BLK = 200
N_ROW_BLKS = N // ROW_BLK
ROW_ITERS = (N_ROW_BLKS + NSUB - 1) // NSUB
EDGE_BLK = 128
N_EDGE_BLKS = E // EDGE_BLK
BLKS_LO = N_EDGE_BLKS // NSUB

_mesh = plsc.VectorSubcoreMesh(core_axis_name="c", subcore_axis_name="s")

_sc_params = pltpu.CompilerParams()
if "needs_layout_passes" in pltpu.CompilerParams.__dataclass_fields__:
    _sc_params = dataclasses.replace(_sc_params, needs_layout_passes=False)
if "use_tc_tiling_on_sc" in pltpu.CompilerParams.__dataclass_fields__:
    _sc_params = dataclasses.replace(_sc_params, use_tc_tiling_on_sc=False)


@functools.partial(
    pl.kernel,
    out_type=(
        jax.ShapeDtypeStruct((2, N, HALF), jnp.float32),
        jax.ShapeDtypeStruct((N,), jnp.float32),
        jax.ShapeDtypeStruct((2, N, HALF), jnp.float32),
    ),
    mesh=_mesh,
    compiler_params=_sc_params,
    scratch_types=[
        pltpu.VMEM_SHARED((N, HALF), jnp.float32),
        pltpu.VMEM_SHARED((N,), jnp.float32),
        pltpu.VMEM((ROW_BLK, HALF), jnp.float32),
        pltpu.VMEM((ROW_BLK, K), jnp.int32),
        pltpu.VMEM((ROW_BLK, K), jnp.float32),
        pltpu.VMEM((4, EDGE_BLK), jnp.int32),
        pltpu.VMEM((4, EDGE_BLK), jnp.int32),
        pltpu.VMEM((2, EDGE_BLK, HALF), jnp.float32),
        pltpu.VMEM((EDGE_BLK,), jnp.float32),
        pltpu.VMEM((N // 5,), jnp.float32),
        pltpu.SemaphoreType.DMA((4,)),
        pltpu.SemaphoreType.DMA((4,)),
        pltpu.SemaphoreType.DMA((2,)),
        pltpu.SemaphoreType.DMA((2,)),
        pltpu.SemaphoreType.DMA((2,)),
    ],
)
def _sc_aggregate(tv_hbm, ti_hbm, src_hbm, dst_hbm, hs_out, deg_out, sf_out,
                  hacc_sh, deg_sh,
                  buf, ti_vm, tv_vm, sidx, didx, stage, ones, zdeg,
                  sem_si, sem_di, sem_g, sem_w, sem_deg):
    c = lax.axis_index("c")
    s = lax.axis_index("s")
    zvec = jnp.zeros((L,), jnp.float32)

    @pl.loop(0, ROW_BLK)
    def _(r):
        for h in range(HALF // L):
            buf[r, pl.ds(h * L, L)] = zvec

    @pl.loop(0, ROW_ITERS)
    def _(i):
        j = i * NSUB + s

        @pl.when(j < N_ROW_BLKS)
        def _():
            pltpu.sync_copy(buf, hacc_sh.at[pl.ds(j * ROW_BLK, ROW_BLK)])

    @pl.when(jnp.logical_and(c == 0, s == 0))
    def _():
        @pl.loop(0, N // 5 // L)
        def _(i):
            zdeg[pl.ds(i * L, L)] = zvec

        @pl.loop(0, 5)
        def _(i):
            pltpu.sync_copy(zdeg, deg_sh.at[pl.ds(i * (N // 5), N // 5)])

    @pl.when(c == 0)
    def _():
        @pl.loop(0, EDGE_BLK // L)
        def _(i):
            ones[pl.ds(i * L, L)] = jnp.ones((L,), jnp.float32)

    coff = c * HALF

    @pl.loop(0, ROW_ITERS)
    def _(i):
        j = i * NSUB + s

        @pl.when(j < N_ROW_BLKS)
        def _():
            base = j * ROW_BLK
            pltpu.sync_copy(ti_hbm.at[pl.ds(base, ROW_BLK)], ti_vm)
            pltpu.sync_copy(tv_hbm.at[pl.ds(base, ROW_BLK)], tv_vm)

            @pl.loop(0, ROW_BLK)
            def _(r):
                row_ids = jnp.full((L,), r, jnp.int32)
                for h in range(K // L):
                    cols = ti_vm[r, pl.ds(h * L, L)] - coff
                    vals = tv_vm[r, pl.ds(h * L, L)]
                    mask = jnp.logical_and(cols >= 0, cols < HALF)
                    plsc.store_scatter(buf, [row_ids, cols], vals, mask=mask)

            pltpu.sync_copy(buf, sf_out.at[c].at[pl.ds(base, ROW_BLK)])

            @pl.loop(0, ROW_BLK)
            def _(r):
                row_ids = jnp.full((L,), r, jnp.int32)
                for h in range(K // L):
                    cols = ti_vm[r, pl.ds(h * L, L)] - coff
                    mask = jnp.logical_and(cols >= 0, cols < HALF)
                    plsc.store_scatter(buf, [row_ids, cols], zvec, mask=mask)

    plsc.subcore_barrier()

    start = s * BLKS_LO + jnp.minimum(s, N_EDGE_BLKS - BLKS_LO * NSUB)
    nb = BLKS_LO + jnp.where(s < N_EDGE_BLKS - BLKS_LO * NSUB, 1, 0)

    def _sidx_cp(i):
        b4 = lax.rem(i, 4)
        base = (start + i) * EDGE_BLK
        return pltpu.make_async_copy(
            src_hbm.at[pl.ds(base, EDGE_BLK)], sidx.at[b4], sem_si.at[b4])

    def _didx_cp(i):
        b4 = lax.rem(i, 4)
        base = (start + i) * EDGE_BLK
        return pltpu.make_async_copy(
            dst_hbm.at[pl.ds(base, EDGE_BLK)], didx.at[b4], sem_di.at[b4])

    def _gather_cp(i):
        b2 = lax.rem(i, 2)
        b4 = lax.rem(i, 4)
        return pltpu.make_async_copy(
            sf_out.at[c].at[sidx.at[b4]], stage.at[b2], sem_g.at[b2])

    def _scatter_cp(i):
        b2 = lax.rem(i, 2)
        b4 = lax.rem(i, 4)
        return pltpu.make_async_copy(
            stage.at[b2], hacc_sh.at[didx.at[b4]], sem_w.at[b2])

    def _deg_cp(i):
        b2 = lax.rem(i, 2)
        b4 = lax.rem(i, 4)
        return pltpu.make_async_copy(
            ones, deg_sh.at[didx.at[b4]], sem_deg.at[b2])

    def idx_start(i):
        _sidx_cp(i).start()
        _didx_cp(i).start()

    def idx_wait(i):
        _sidx_cp(i).wait()
        _didx_cp(i).wait()

    def scatter_start(i):
        _scatter_cp(i).start(add=True)

        @pl.when(c == 0)
        def _():
            _deg_cp(i).start(add=True)

    def scatter_wait(i):
        _scatter_cp(i).wait()

        @pl.when(c == 0)
        def _():
            _deg_cp(i).wait()

    idx_start(0)
    idx_start(1)
    idx_wait(0)
    _gather_cp(0).start()

    @pl.loop(1, nb)
    def _(i):
        @pl.when(i + 1 < nb)
        def _():
            idx_start(i + 1)

        idx_wait(i)

        @pl.when(i >= 2)
        def _():
            scatter_wait(i - 2)

        _gather_cp(i).start()
        _gather_cp(i - 1).wait()
        scatter_start(i - 1)

    _gather_cp(nb - 1).wait()
    scatter_start(nb - 1)
    scatter_wait(nb - 2)
    scatter_wait(nb - 1)

    plsc.subcore_barrier()

    @pl.loop(0, ROW_ITERS)
    def _(i):
        j = i * NSUB + s

        @pl.when(j < N_ROW_BLKS)
        def _():
            base = j * ROW_BLK
            pltpu.sync_copy(hacc_sh.at[pl.ds(base, ROW_BLK)],
                            hs_out.at[c].at[pl.ds(base, ROW_BLK)])

    @pl.when(jnp.logical_and(c == 0, s == 0))
    def _():
        pltpu.sync_copy(deg_sh, deg_out)


R_TC = 1000
_TC_GRID = N // R_TC


def _tc_body(feat_ref, h0_ref, h1_ref, w_ref, ws_ref, wn_ref, b_ref, out_ref):
    w = w_ref[...]
    h0w = h0_ref[...] * w
    h1w = h1_ref[...] * w
    dn = (((1,), (1,)), ((), ()))
    acc = lax.dot_general(feat_ref[...], ws_ref[...], dn,
                          preferred_element_type=jnp.float32,
                          precision=lax.Precision.HIGHEST)
    wn = wn_ref[...]
    acc += lax.dot_general(h0w, wn[:, :HALF], dn,
                           preferred_element_type=jnp.float32,
                           precision=lax.Precision.HIGHEST)
    acc += lax.dot_general(h1w, wn[:, HALF:], dn,
                           preferred_element_type=jnp.float32,
                           precision=lax.Precision.HIGHEST)
    out_ref[...] = acc + b_ref[...]


def _tc_combine(feat, h0, h1, winv, W_self, W_neigh, b):
    return pl.pallas_call(
        _tc_body,
        grid=(_TC_GRID,),
        in_specs=[
            pl.BlockSpec((R_TC, D), lambda i: (i, 0)),
            pl.BlockSpec((R_TC, HALF), lambda i: (i, 0)),
            pl.BlockSpec((R_TC, HALF), lambda i: (i, 0)),
            pl.BlockSpec((R_TC, 1), lambda i: (i, 0)),
            pl.BlockSpec((D, D), lambda i: (0, 0)),
            pl.BlockSpec((D, D), lambda i: (0, 0)),
            pl.BlockSpec((1, D), lambda i: (0, 0)),
        ],
        out_specs=pl.BlockSpec((R_TC, D), lambda i: (i, 0)),
        out_shape=jax.ShapeDtypeStruct((N, D), jnp.float32),
    )(feat, h0, h1, winv, W_self, W_neigh, b)


def kernel(feat, topk_values, topk_indices, edge_index, W_neigh, W_self, b_self):
    ti = topk_indices.astype(jnp.int32)
    src = edge_index[0].astype(jnp.int32)
    dst = edge_index[1].astype(jnp.int32)
    hs, deg, _ = _sc_aggregate(topk_values.astype(jnp.float32), ti, src, dst)
    winv = (1.0 / jnp.maximum(deg, 1.0))[:, None]
    return _tc_combine(feat, hs[0], hs[1], winv,
                       W_self, W_neigh, b_self.reshape(1, D))


# speedup vs baseline: 24.8693x; 1.1518x over previous
import dataclasses
import functools

import jax
import jax.numpy as jnp
from jax import lax
from jax.experimental import pallas as pl
from jax.experimental.pallas import tpu as pltpu
from jax.experimental.pallas import tpu_sc as plsc

N = 10000
E = 320000
D = 128
K = 32
HALF = 64
L = 16
NSUB = 16
ROW_BLK = 200
N_ROW_BLKS = N // ROW_BLK
ROW_ITERS = (N_ROW_BLKS + NSUB - 1) // NSUB
EDGE_BLK = 128
N_EDGE_BLKS = E // EDGE_BLK
BLKS_LO = N_EDGE_BLKS // NSUB

_mesh = plsc.VectorSubcoreMesh(core_axis_name="c", subcore_axis_name="s")

_sc_params = pltpu.CompilerParams()
if "needs_layout_passes" in pltpu.CompilerParams.__dataclass_fields__:
    _sc_params = dataclasses.replace(_sc_params, needs_layout_passes=False)
if "use_tc_tiling_on_sc" in pltpu.CompilerParams.__dataclass_fields__:
    _sc_params = dataclasses.replace(_sc_params, use_tc_tiling_on_sc=False)


@functools.partial(
    pl.kernel,
    out_type=(
        jax.ShapeDtypeStruct((2, N, HALF), jnp.float32),
        jax.ShapeDtypeStruct((N,), jnp.float32),
        jax.ShapeDtypeStruct((2, N, HALF), jnp.float32),
    ),
    mesh=_mesh,
    compiler_params=_sc_params,
    scratch_types=[
        pltpu.VMEM_SHARED((N, HALF), jnp.float32),
        pltpu.VMEM_SHARED((N,), jnp.float32),
        pltpu.VMEM((ROW_BLK, HALF), jnp.float32),
        pltpu.VMEM((ROW_BLK, K), jnp.int32),
        pltpu.VMEM((ROW_BLK, K), jnp.float32),
        pltpu.VMEM((4, EDGE_BLK), jnp.int32),
        pltpu.VMEM((4, EDGE_BLK), jnp.int32),
        pltpu.VMEM((2, EDGE_BLK, HALF), jnp.float32),
        pltpu.VMEM((EDGE_BLK,), jnp.float32),
        pltpu.VMEM((N // 5,), jnp.float32),
        pltpu.SemaphoreType.DMA((4,)),
        pltpu.SemaphoreType.DMA((4,)),
        pltpu.SemaphoreType.DMA((2,)),
        pltpu.SemaphoreType.DMA((2,)),
        pltpu.SemaphoreType.DMA((2,)),
    ],
)
def _sc_aggregate(tv_hbm, ti_hbm, ei_hbm, hs_out, deg_out, sf_out,
                  hacc_sh, deg_sh,
                  buf, ti_vm, tv_vm, sidx, didx, stage, ones, zdeg,
                  sem_si, sem_di, sem_g, sem_w, sem_deg):
    src_hbm = ei_hbm.at[0]
    dst_hbm = ei_hbm.at[1]
    c = lax.axis_index("c")
    s = lax.axis_index("s")
    zvec = jnp.zeros((L,), jnp.float32)

    @pl.loop(0, ROW_BLK)
    def _(r):
        for h in range(HALF // L):
            buf[r, pl.ds(h * L, L)] = zvec

    @pl.loop(0, ROW_ITERS)
    def _(i):
        j = i * NSUB + s

        @pl.when(j < N_ROW_BLKS)
        def _():
            pltpu.sync_copy(buf, hacc_sh.at[pl.ds(j * ROW_BLK, ROW_BLK)])

    @pl.when(jnp.logical_and(c == 0, s == 0))
    def _():
        @pl.loop(0, N // 5 // L)
        def _(i):
            zdeg[pl.ds(i * L, L)] = zvec

        @pl.loop(0, 5)
        def _(i):
            pltpu.sync_copy(zdeg, deg_sh.at[pl.ds(i * (N // 5), N // 5)])

    @pl.when(c == 0)
    def _():
        @pl.loop(0, EDGE_BLK // L)
        def _(i):
            ones[pl.ds(i * L, L)] = jnp.ones((L,), jnp.float32)

    coff = c * HALF

    @pl.loop(0, ROW_ITERS)
    def _(i):
        j = i * NSUB + s

        @pl.when(j < N_ROW_BLKS)
        def _():
            base = j * ROW_BLK
            pltpu.sync_copy(ti_hbm.at[pl.ds(base, ROW_BLK)], ti_vm)
            pltpu.sync_copy(tv_hbm.at[pl.ds(base, ROW_BLK)], tv_vm)

            @pl.loop(0, ROW_BLK)
            def _(r):
                row_ids = jnp.full((L,), r, jnp.int32)
                for h in range(K // L):
                    cols = ti_vm[r, pl.ds(h * L, L)] - coff
                    vals = tv_vm[r, pl.ds(h * L, L)]
                    mask = jnp.logical_and(cols >= 0, cols < HALF)
                    plsc.store_scatter(buf, [row_ids, cols], vals, mask=mask)

            pltpu.sync_copy(buf, sf_out.at[c].at[pl.ds(base, ROW_BLK)])

            @pl.when(i < ROW_ITERS - 1)
            def _():
                pltpu.sync_copy(hacc_sh.at[pl.ds(s * ROW_BLK, ROW_BLK)], buf)

    plsc.subcore_barrier()

    start = s * BLKS_LO + jnp.minimum(s, N_EDGE_BLKS - BLKS_LO * NSUB)
    nb = BLKS_LO + jnp.where(s < N_EDGE_BLKS - BLKS_LO * NSUB, 1, 0)

    def _sidx_cp(i):
        b4 = lax.rem(i, 4)
        base = (start + i) * EDGE_BLK
        return pltpu.make_async_copy(
            src_hbm.at[pl.ds(base, EDGE_BLK)], sidx.at[b4], sem_si.at[b4])

    def _didx_cp(i):
        b4 = lax.rem(i, 4)
        base = (start + i) * EDGE_BLK
        return pltpu.make_async_copy(
            dst_hbm.at[pl.ds(base, EDGE_BLK)], didx.at[b4], sem_di.at[b4])

    def _gather_cp(i):
        b2 = lax.rem(i, 2)
        b4 = lax.rem(i, 4)
        return pltpu.make_async_copy(
            sf_out.at[c].at[sidx.at[b4]], stage.at[b2], sem_g.at[b2])

    def _scatter_cp(i):
        b2 = lax.rem(i, 2)
        b4 = lax.rem(i, 4)
        return pltpu.make_async_copy(
            stage.at[b2], hacc_sh.at[didx.at[b4]], sem_w.at[b2])

    def _deg_cp(i):
        b2 = lax.rem(i, 2)
        b4 = lax.rem(i, 4)
        return pltpu.make_async_copy(
            ones, deg_sh.at[didx.at[b4]], sem_deg.at[b2])

    def idx_start(i):
        _sidx_cp(i).start()
        _didx_cp(i).start()

    def idx_wait(i):
        _sidx_cp(i).wait()
        _didx_cp(i).wait()

    def scatter_start(i):
        _scatter_cp(i).start(add=True)

        @pl.when(c == 0)
        def _():
            _deg_cp(i).start(add=True)

    def scatter_wait(i):
        _scatter_cp(i).wait()

        @pl.when(c == 0)
        def _():
            _deg_cp(i).wait()

    idx_start(0)
    idx_start(1)
    idx_wait(0)
    _gather_cp(0).start()

    @pl.loop(1, nb)
    def _(i):
        @pl.when(i + 1 < nb)
        def _():
            idx_start(i + 1)

        idx_wait(i)

        @pl.when(i >= 2)
        def _():
            scatter_wait(i - 2)

        _gather_cp(i).start()
        _gather_cp(i - 1).wait()
        scatter_start(i - 1)

    _gather_cp(nb - 1).wait()
    scatter_start(nb - 1)
    scatter_wait(nb - 2)
    scatter_wait(nb - 1)

    plsc.subcore_barrier()

    @pl.loop(0, ROW_ITERS)
    def _(i):
        j = i * NSUB + s

        @pl.when(j < N_ROW_BLKS)
        def _():
            base = j * ROW_BLK
            pltpu.sync_copy(hacc_sh.at[pl.ds(base, ROW_BLK)],
                            hs_out.at[c].at[pl.ds(base, ROW_BLK)])

    @pl.when(jnp.logical_and(c == 0, s == 0))
    def _():
        pltpu.sync_copy(deg_sh, deg_out)


R_TC = 1000
_TC_GRID = N // R_TC


def _tc_body(feat_ref, h0_ref, h1_ref, deg_ref, ws_ref, wn_ref, b_ref, out_ref):
    w = 1.0 / jnp.maximum(deg_ref[...], 1.0)
    h0w = h0_ref[0] * w
    h1w = h1_ref[0] * w
    dn = (((1,), (1,)), ((), ()))
    acc = lax.dot_general(feat_ref[...], ws_ref[...], dn,
                          preferred_element_type=jnp.float32,
                          precision=lax.Precision.HIGHEST)
    wn = wn_ref[...]
    acc += lax.dot_general(h0w, wn[:, :HALF], dn,
                           preferred_element_type=jnp.float32,
                           precision=lax.Precision.HIGHEST)
    acc += lax.dot_general(h1w, wn[:, HALF:], dn,
                           preferred_element_type=jnp.float32,
                           precision=lax.Precision.HIGHEST)
    out_ref[...] = acc + b_ref[...]


def _tc_combine(feat, hs, deg2d, W_self, W_neigh, b):
    return pl.pallas_call(
        _tc_body,
        grid=(_TC_GRID,),
        in_specs=[
            pl.BlockSpec((R_TC, D), lambda i: (i, 0)),
            pl.BlockSpec((1, R_TC, HALF), lambda i: (0, i, 0)),
            pl.BlockSpec((1, R_TC, HALF), lambda i: (1, i, 0)),
            pl.BlockSpec((R_TC, 1), lambda i: (i, 0)),
            pl.BlockSpec((D, D), lambda i: (0, 0)),
            pl.BlockSpec((D, D), lambda i: (0, 0)),
            pl.BlockSpec((1, D), lambda i: (0, 0)),
        ],
        out_specs=pl.BlockSpec((R_TC, D), lambda i: (i, 0)),
        out_shape=jax.ShapeDtypeStruct((N, D), jnp.float32),
    )(feat, hs, hs, deg2d, W_self, W_neigh, b)


def kernel(feat, topk_values, topk_indices, edge_index, W_neigh, W_self, b_self):
    ti = topk_indices.astype(jnp.int32)
    ei = edge_index.astype(jnp.int32)
    hs, deg, _ = _sc_aggregate(topk_values.astype(jnp.float32), ti, ei)
    return _tc_combine(feat, hs, deg[:, None],
                       W_self, W_neigh, b_self.reshape(1, D))


# speedup vs baseline: 27.2113x; 1.0942x over previous
import dataclasses
import functools

import jax
import jax.numpy as jnp
from jax import lax
from jax.experimental import pallas as pl
from jax.experimental.pallas import tpu as pltpu
from jax.experimental.pallas import tpu_sc as plsc

N = 10000
E = 320000
D = 128
K = 32
HALF = 64
L = 16
NSUB = 16
ROW_BLK = 200
N_ROW_BLKS = N // ROW_BLK
ROW_ITERS = (N_ROW_BLKS + NSUB - 1) // NSUB
EDGE_BLK = 128
N_EDGE_BLKS = E // EDGE_BLK
BLKS_LO = N_EDGE_BLKS // NSUB

_mesh = plsc.VectorSubcoreMesh(core_axis_name="c", subcore_axis_name="s")

_sc_params = pltpu.CompilerParams()
if "needs_layout_passes" in pltpu.CompilerParams.__dataclass_fields__:
    _sc_params = dataclasses.replace(_sc_params, needs_layout_passes=False)
if "use_tc_tiling_on_sc" in pltpu.CompilerParams.__dataclass_fields__:
    _sc_params = dataclasses.replace(_sc_params, use_tc_tiling_on_sc=False)


@functools.partial(
    pl.kernel,
    out_type=(
        jax.ShapeDtypeStruct((2, N, HALF), jnp.float32),
        jax.ShapeDtypeStruct((N,), jnp.float32),
        jax.ShapeDtypeStruct((2, N, HALF), jnp.float32),
    ),
    mesh=_mesh,
    compiler_params=_sc_params,
    scratch_types=[
        pltpu.VMEM_SHARED((N, HALF), jnp.float32),
        pltpu.VMEM_SHARED((N,), jnp.float32),
        pltpu.VMEM((ROW_BLK, HALF), jnp.float32),
        pltpu.VMEM((ROW_BLK, K), jnp.int32),
        pltpu.VMEM((ROW_BLK, K), jnp.float32),
        pltpu.VMEM((4, EDGE_BLK), jnp.int32),
        pltpu.VMEM((4, EDGE_BLK), jnp.int32),
        pltpu.VMEM((2, EDGE_BLK, HALF), jnp.float32),
        pltpu.VMEM((EDGE_BLK,), jnp.float32),
        pltpu.VMEM((N // 5,), jnp.float32),
        pltpu.SemaphoreType.DMA((4,)),
        pltpu.SemaphoreType.DMA((4,)),
        pltpu.SemaphoreType.DMA((2,)),
        pltpu.SemaphoreType.DMA((2,)),
        pltpu.SemaphoreType.DMA((2,)),
    ],
)
def _sc_aggregate(tv_hbm, ti_hbm, ei_hbm, hs_out, deg_out, sf_out,
                  hacc_sh, deg_sh,
                  buf, ti_vm, tv_vm, sidx, didx, stage, ones, zdeg,
                  sem_si, sem_di, sem_g, sem_w, sem_deg):
    src_hbm = ei_hbm.at[0]
    dst_hbm = ei_hbm.at[1]
    c = lax.axis_index("c")
    s = lax.axis_index("s")
    zvec = jnp.zeros((L,), jnp.float32)

    @pl.loop(0, ROW_BLK)
    def _(r):
        for h in range(HALF // L):
            buf[r, pl.ds(h * L, L)] = zvec

    @pl.loop(0, ROW_ITERS)
    def _(i):
        j = i * NSUB + s

        @pl.when(j < N_ROW_BLKS)
        def _():
            pltpu.sync_copy(buf, hacc_sh.at[pl.ds(j * ROW_BLK, ROW_BLK)])

    @pl.when(jnp.logical_and(c == 0, s == 0))
    def _():
        @pl.loop(0, N // 5 // L)
        def _(i):
            zdeg[pl.ds(i * L, L)] = zvec

        @pl.loop(0, 5)
        def _(i):
            pltpu.sync_copy(zdeg, deg_sh.at[pl.ds(i * (N // 5), N // 5)])

    @pl.when(c == 0)
    def _():
        @pl.loop(0, EDGE_BLK // L)
        def _(i):
            ones[pl.ds(i * L, L)] = jnp.ones((L,), jnp.float32)

    coff = c * HALF

    @pl.loop(0, ROW_ITERS)
    def _(i):
        j = i * NSUB + s

        @pl.when(j < N_ROW_BLKS)
        def _():
            base = j * ROW_BLK
            pltpu.sync_copy(ti_hbm.at[pl.ds(base, ROW_BLK)], ti_vm)
            pltpu.sync_copy(tv_hbm.at[pl.ds(base, ROW_BLK)], tv_vm)

            @pl.loop(0, ROW_BLK)
            def _(r):
                row_ids = jnp.full((L,), r, jnp.int32)
                for h in range(K // L):
                    cols = ti_vm[r, pl.ds(h * L, L)] - coff
                    vals = tv_vm[r, pl.ds(h * L, L)]
                    mask = jnp.logical_and(cols >= 0, cols < HALF)
                    plsc.store_scatter(buf, [row_ids, cols], vals, mask=mask)

            pltpu.sync_copy(buf, sf_out.at[c].at[pl.ds(base, ROW_BLK)])

            @pl.when(i < ROW_ITERS - 1)
            def _():
                pltpu.sync_copy(hacc_sh.at[pl.ds(s * ROW_BLK, ROW_BLK)], buf)

    plsc.subcore_barrier()

    start = s * BLKS_LO + jnp.minimum(s, N_EDGE_BLKS - BLKS_LO * NSUB)
    nb = BLKS_LO + jnp.where(s < N_EDGE_BLKS - BLKS_LO * NSUB, 1, 0)

    def _sidx_cp(i):
        b4 = lax.rem(i, 4)
        base = (start + i) * EDGE_BLK
        return pltpu.make_async_copy(
            src_hbm.at[pl.ds(base, EDGE_BLK)], sidx.at[b4], sem_si.at[b4])

    def _didx_cp(i):
        b4 = lax.rem(i, 4)
        base = (start + i) * EDGE_BLK
        return pltpu.make_async_copy(
            dst_hbm.at[pl.ds(base, EDGE_BLK)], didx.at[b4], sem_di.at[b4])

    def _gather_cp(i):
        b2 = lax.rem(i, 2)
        b4 = lax.rem(i, 4)
        return pltpu.make_async_copy(
            sf_out.at[c].at[sidx.at[b4]], stage.at[b2], sem_g.at[b2])

    def _scatter_cp(i):
        b2 = lax.rem(i, 2)
        b4 = lax.rem(i, 4)
        return pltpu.make_async_copy(
            stage.at[b2], hacc_sh.at[didx.at[b4]], sem_w.at[b2])

    def _deg_cp(i):
        b2 = lax.rem(i, 2)
        b4 = lax.rem(i, 4)
        return pltpu.make_async_copy(
            ones, deg_sh.at[didx.at[b4]], sem_deg.at[b2])

    def idx_start(i):
        _sidx_cp(i).start()
        _didx_cp(i).start()

    def idx_wait(i):
        _sidx_cp(i).wait()
        _didx_cp(i).wait()

    def scatter_start(i):
        _scatter_cp(i).start(add=True)

        @pl.when(c == 0)
        def _():
            _deg_cp(i).start(add=True)

    def scatter_wait(i):
        _scatter_cp(i).wait()

        @pl.when(c == 0)
        def _():
            _deg_cp(i).wait()

    idx_start(0)
    idx_start(1)
    idx_wait(0)
    _gather_cp(0).start()

    @pl.loop(1, nb)
    def _(i):
        @pl.when(i + 1 < nb)
        def _():
            idx_start(i + 1)

        idx_wait(i)

        @pl.when(i >= 2)
        def _():
            scatter_wait(i - 2)

        _gather_cp(i).start()
        _gather_cp(i - 1).wait()
        scatter_start(i - 1)

    _gather_cp(nb - 1).wait()
    scatter_start(nb - 1)
    scatter_wait(nb - 2)
    scatter_wait(nb - 1)

    plsc.subcore_barrier()

    @pl.loop(0, ROW_ITERS)
    def _(i):
        j = i * NSUB + s

        @pl.when(j < N_ROW_BLKS)
        def _():
            base = j * ROW_BLK
            pltpu.sync_copy(hacc_sh.at[pl.ds(base, ROW_BLK)],
                            hs_out.at[c].at[pl.ds(base, ROW_BLK)])

    @pl.when(jnp.logical_and(c == 0, s == 0))
    def _():
        pltpu.sync_copy(deg_sh, deg_out)


R_TC = 2000
_TC_GRID = N // R_TC


def _tc_body(feat_ref, h0_ref, h1_ref, deg_ref, ws_ref, wn_ref, b_ref, out_ref):
    w = 1.0 / jnp.maximum(deg_ref[...], 1.0)
    h0w = h0_ref[0] * w
    h1w = h1_ref[0] * w
    dn = (((1,), (1,)), ((), ()))
    acc = lax.dot_general(feat_ref[...], ws_ref[...], dn,
                          preferred_element_type=jnp.float32)
    wn = wn_ref[...]
    acc += lax.dot_general(h0w, wn[:, :HALF], dn,
                           preferred_element_type=jnp.float32)
    acc += lax.dot_general(h1w, wn[:, HALF:], dn,
                           preferred_element_type=jnp.float32)
    out_ref[...] = acc + b_ref[...]


def _tc_combine(feat, hs, deg2d, W_self, W_neigh, b):
    return pl.pallas_call(
        _tc_body,
        grid=(_TC_GRID,),
        in_specs=[
            pl.BlockSpec((R_TC, D), lambda i: (i, 0)),
            pl.BlockSpec((1, R_TC, HALF), lambda i: (0, i, 0)),
            pl.BlockSpec((1, R_TC, HALF), lambda i: (1, i, 0)),
            pl.BlockSpec((R_TC, 1), lambda i: (i, 0)),
            pl.BlockSpec((D, D), lambda i: (0, 0)),
            pl.BlockSpec((D, D), lambda i: (0, 0)),
            pl.BlockSpec((D,), lambda i: (0,)),
        ],
        out_specs=pl.BlockSpec((R_TC, D), lambda i: (i, 0)),
        out_shape=jax.ShapeDtypeStruct((N, D), jnp.float32),
    )(feat, hs, hs, deg2d, W_self, W_neigh, b)


def kernel(feat, topk_values, topk_indices, edge_index, W_neigh, W_self, b_self):
    ti = topk_indices.astype(jnp.int32)
    ei = edge_index.astype(jnp.int32)
    hs, deg, _ = _sc_aggregate(topk_values.astype(jnp.float32), ti, ei)
    return _tc_combine(feat, hs, deg[:, None], W_self, W_neigh, b_self)


# speedup vs baseline: 28.0361x; 1.0303x over previous
import dataclasses
import functools

import jax
import jax.numpy as jnp
from jax import lax
from jax.experimental import pallas as pl
from jax.experimental.pallas import tpu as pltpu
from jax.experimental.pallas import tpu_sc as plsc

N = 10000
E = 320000
D = 128
K = 32
HALF = 64
L = 16
NSUB = 16
ROW_BLK = 200
N_ROW_BLKS = N // ROW_BLK
ROW_ITERS = (N_ROW_BLKS + NSUB - 1) // NSUB
EDGE_BLK = 128
N_EDGE_BLKS = E // EDGE_BLK
BLKS_LO = N_EDGE_BLKS // NSUB

_mesh = plsc.VectorSubcoreMesh(core_axis_name="c", subcore_axis_name="s")

_sc_params = pltpu.CompilerParams()
if "needs_layout_passes" in pltpu.CompilerParams.__dataclass_fields__:
    _sc_params = dataclasses.replace(_sc_params, needs_layout_passes=False)
if "use_tc_tiling_on_sc" in pltpu.CompilerParams.__dataclass_fields__:
    _sc_params = dataclasses.replace(_sc_params, use_tc_tiling_on_sc=False)


@functools.partial(
    pl.kernel,
    out_type=(
        jax.ShapeDtypeStruct((N, D), jnp.float32),
        jax.ShapeDtypeStruct((N,), jnp.float32),
        jax.ShapeDtypeStruct((2, N, HALF), jnp.float32),
    ),
    mesh=_mesh,
    compiler_params=_sc_params,
    scratch_types=[
        pltpu.VMEM_SHARED((N, HALF), jnp.float32),
        pltpu.VMEM_SHARED((N,), jnp.float32),
        pltpu.VMEM((ROW_BLK, HALF), jnp.float32),
        pltpu.VMEM((ROW_BLK, K), jnp.int32),
        pltpu.VMEM((ROW_BLK, K), jnp.float32),
        pltpu.VMEM((4, EDGE_BLK), jnp.int32),
        pltpu.VMEM((4, EDGE_BLK), jnp.int32),
        pltpu.VMEM((2, EDGE_BLK, HALF), jnp.float32),
        pltpu.VMEM((EDGE_BLK,), jnp.float32),
        pltpu.VMEM((N // 5,), jnp.float32),
        pltpu.SemaphoreType.DMA((4,)),
        pltpu.SemaphoreType.DMA((4,)),
        pltpu.SemaphoreType.DMA((2,)),
        pltpu.SemaphoreType.DMA((2,)),
        pltpu.SemaphoreType.DMA((2,)),
    ],
)
def _sc_aggregate(tv_hbm, ti_hbm, ei_hbm, hs_out, deg_out, sf_out,
                  hacc_sh, deg_sh,
                  buf, ti_vm, tv_vm, sidx, didx, stage, ones, zdeg,
                  sem_si, sem_di, sem_g, sem_w, sem_deg):
    src_hbm = ei_hbm.at[0]
    dst_hbm = ei_hbm.at[1]
    c = lax.axis_index("c")
    s = lax.axis_index("s")
    zvec = jnp.zeros((L,), jnp.float32)

    @pl.loop(0, ROW_BLK)
    def _(r):
        for h in range(HALF // L):
            buf[r, pl.ds(h * L, L)] = zvec

    @pl.loop(0, ROW_ITERS)
    def _(i):
        j = i * NSUB + s

        @pl.when(j < N_ROW_BLKS)
        def _():
            pltpu.sync_copy(buf, hacc_sh.at[pl.ds(j * ROW_BLK, ROW_BLK)])

    @pl.when(jnp.logical_and(c == 0, s == 0))
    def _():
        @pl.loop(0, N // 5 // L)
        def _(i):
            zdeg[pl.ds(i * L, L)] = zvec

        @pl.loop(0, 5)
        def _(i):
            pltpu.sync_copy(zdeg, deg_sh.at[pl.ds(i * (N // 5), N // 5)])

    @pl.when(c == 0)
    def _():
        @pl.loop(0, EDGE_BLK // L)
        def _(i):
            ones[pl.ds(i * L, L)] = jnp.ones((L,), jnp.float32)

    coff = c * HALF

    @pl.loop(0, ROW_ITERS)
    def _(i):
        j = i * NSUB + s

        @pl.when(j < N_ROW_BLKS)
        def _():
            base = j * ROW_BLK
            pltpu.sync_copy(ti_hbm.at[pl.ds(base, ROW_BLK)], ti_vm)
            pltpu.sync_copy(tv_hbm.at[pl.ds(base, ROW_BLK)], tv_vm)

            @pl.loop(0, ROW_BLK)
            def _(r):
                row_ids = jnp.full((L,), r, jnp.int32)
                for h in range(K // L):
                    cols = ti_vm[r, pl.ds(h * L, L)] - coff
                    vals = tv_vm[r, pl.ds(h * L, L)]
                    mask = jnp.logical_and(cols >= 0, cols < HALF)
                    plsc.store_scatter(buf, [row_ids, cols], vals, mask=mask)

            pltpu.sync_copy(buf, sf_out.at[c].at[pl.ds(base, ROW_BLK)])

            @pl.when(i < ROW_ITERS - 1)
            def _():
                pltpu.sync_copy(hacc_sh.at[pl.ds(s * ROW_BLK, ROW_BLK)], buf)

    plsc.subcore_barrier()

    start = s * BLKS_LO + jnp.minimum(s, N_EDGE_BLKS - BLKS_LO * NSUB)
    nb = BLKS_LO + jnp.where(s < N_EDGE_BLKS - BLKS_LO * NSUB, 1, 0)

    def _sidx_cp(i):
        b4 = lax.rem(i, 4)
        base = (start + i) * EDGE_BLK
        return pltpu.make_async_copy(
            src_hbm.at[pl.ds(base, EDGE_BLK)], sidx.at[b4], sem_si.at[b4])

    def _didx_cp(i):
        b4 = lax.rem(i, 4)
        base = (start + i) * EDGE_BLK
        return pltpu.make_async_copy(
            dst_hbm.at[pl.ds(base, EDGE_BLK)], didx.at[b4], sem_di.at[b4])

    def _gather_cp(i):
        b2 = lax.rem(i, 2)
        b4 = lax.rem(i, 4)
        return pltpu.make_async_copy(
            sf_out.at[c].at[sidx.at[b4]], stage.at[b2], sem_g.at[b2])

    def _scatter_cp(i):
        b2 = lax.rem(i, 2)
        b4 = lax.rem(i, 4)
        return pltpu.make_async_copy(
            stage.at[b2], hacc_sh.at[didx.at[b4]], sem_w.at[b2])

    def _deg_cp(i):
        b2 = lax.rem(i, 2)
        b4 = lax.rem(i, 4)
        return pltpu.make_async_copy(
            ones, deg_sh.at[didx.at[b4]], sem_deg.at[b2])

    def idx_start(i):
        _sidx_cp(i).start()
        _didx_cp(i).start()

    def idx_wait(i):
        _sidx_cp(i).wait()
        _didx_cp(i).wait()

    def scatter_start(i):
        _scatter_cp(i).start(add=True)

        @pl.when(c == 0)
        def _():
            _deg_cp(i).start(add=True)

    def scatter_wait(i):
        _scatter_cp(i).wait()

        @pl.when(c == 0)
        def _():
            _deg_cp(i).wait()

    idx_start(0)
    idx_start(1)
    idx_wait(0)
    _gather_cp(0).start()

    @pl.loop(1, nb)
    def _(i):
        @pl.when(i + 1 < nb)
        def _():
            idx_start(i + 1)

        idx_wait(i)

        @pl.when(i >= 2)
        def _():
            scatter_wait(i - 2)

        _gather_cp(i).start()
        _gather_cp(i - 1).wait()
        scatter_start(i - 1)

    _gather_cp(nb - 1).wait()
    scatter_start(nb - 1)
    scatter_wait(nb - 2)
    scatter_wait(nb - 1)

    plsc.subcore_barrier()

    @pl.loop(0, ROW_ITERS)
    def _(i):
        j = i * NSUB + s

        @pl.when(j < N_ROW_BLKS)
        def _():
            base = j * ROW_BLK
            pltpu.sync_copy(hacc_sh.at[pl.ds(base, ROW_BLK)],
                            hs_out.at[pl.ds(base, ROW_BLK), pl.ds(coff, HALF)])

    @pl.when(jnp.logical_and(c == 0, s == 0))
    def _():
        pltpu.sync_copy(deg_sh, deg_out)


R_TC = 2000
_TC_GRID = N // R_TC


def _tc_body(feat_ref, h_ref, deg_ref, ws_ref, wn_ref, b_ref, out_ref):
    w = 1.0 / jnp.maximum(deg_ref[...], 1.0)
    hw = h_ref[...] * w
    dn = (((1,), (1,)), ((), ()))
    acc = lax.dot_general(feat_ref[...], ws_ref[...], dn,
                          preferred_element_type=jnp.float32)
    acc += lax.dot_general(hw, wn_ref[...], dn,
                           preferred_element_type=jnp.float32)
    out_ref[...] = acc + b_ref[...]


def _tc_combine(feat, hs, deg2d, W_self, W_neigh, b):
    return pl.pallas_call(
        _tc_body,
        grid=(_TC_GRID,),
        in_specs=[
            pl.BlockSpec((R_TC, D), lambda i: (i, 0)),
            pl.BlockSpec((R_TC, D), lambda i: (i, 0)),
            pl.BlockSpec((R_TC, 1), lambda i: (i, 0)),
            pl.BlockSpec((D, D), lambda i: (0, 0)),
            pl.BlockSpec((D, D), lambda i: (0, 0)),
            pl.BlockSpec((D,), lambda i: (0,)),
        ],
        out_specs=pl.BlockSpec((R_TC, D), lambda i: (i, 0)),
        out_shape=jax.ShapeDtypeStruct((N, D), jnp.float32),
    )(feat, hs, deg2d, W_self, W_neigh, b)


def kernel(feat, topk_values, topk_indices, edge_index, W_neigh, W_self, b_self):
    ti = topk_indices.astype(jnp.int32)
    ei = edge_index.astype(jnp.int32)
    hs, deg, _ = _sc_aggregate(topk_values.astype(jnp.float32), ti, ei)
    return _tc_combine(feat, hs, deg[:, None], W_self, W_neigh, b_self)


# speedup vs baseline: 29.3753x; 1.0478x over previous
import dataclasses
import functools

import jax
import jax.numpy as jnp
from jax import lax
from jax.experimental import pallas as pl
from jax.experimental.pallas import tpu as pltpu
from jax.experimental.pallas import tpu_sc as plsc

N = 10000
E = 320000
D = 128
K = 32
HALF = 64
L = 16
NSUB = 16
ROW_BLK = 200
N_ROW_BLKS = N // ROW_BLK
ROW_ITERS = (N_ROW_BLKS + NSUB - 1) // NSUB
EDGE_BLK = 128
N_EDGE_BLKS = E // EDGE_BLK
BLKS_LO = N_EDGE_BLKS // NSUB

_mesh = plsc.VectorSubcoreMesh(core_axis_name="c", subcore_axis_name="s")

_sc_params = pltpu.CompilerParams()
if "needs_layout_passes" in pltpu.CompilerParams.__dataclass_fields__:
    _sc_params = dataclasses.replace(_sc_params, needs_layout_passes=False)
if "use_tc_tiling_on_sc" in pltpu.CompilerParams.__dataclass_fields__:
    _sc_params = dataclasses.replace(_sc_params, use_tc_tiling_on_sc=False)


@functools.partial(
    pl.kernel,
    out_type=(
        jax.ShapeDtypeStruct((N, D), jnp.float32),
        jax.ShapeDtypeStruct((N,), jnp.float32),
        jax.ShapeDtypeStruct((2, N, HALF), jnp.float32),
    ),
    mesh=_mesh,
    compiler_params=_sc_params,
    scratch_types=[
        pltpu.VMEM_SHARED((N, HALF), jnp.float32),
        pltpu.VMEM_SHARED((N,), jnp.float32),
        pltpu.VMEM((2, ROW_BLK, HALF), jnp.float32),
        pltpu.VMEM((2, ROW_BLK, K), jnp.int32),
        pltpu.VMEM((2, ROW_BLK, K), jnp.float32),
        pltpu.VMEM((4, EDGE_BLK), jnp.int32),
        pltpu.VMEM((4, EDGE_BLK), jnp.int32),
        pltpu.VMEM((2, EDGE_BLK, HALF), jnp.float32),
        pltpu.VMEM((EDGE_BLK,), jnp.float32),
        pltpu.VMEM((N // 5,), jnp.float32),
        pltpu.SemaphoreType.DMA((4,)),
        pltpu.SemaphoreType.DMA((4,)),
        pltpu.SemaphoreType.DMA((2,)),
        pltpu.SemaphoreType.DMA((2,)),
        pltpu.SemaphoreType.DMA((2,)),
        pltpu.SemaphoreType.DMA((2,)),
        pltpu.SemaphoreType.DMA((2,)),
        pltpu.SemaphoreType.DMA((2,)),
        pltpu.SemaphoreType.DMA,
    ],
)
def _sc_aggregate(tv_hbm, ti_hbm, ei_hbm, hs_out, deg_out, sf_out,
                  hacc_sh, deg_sh,
                  buf, ti_vm, tv_vm, sidx, didx, stage, ones, zdeg,
                  sem_si, sem_di, sem_g, sem_w, sem_deg,
                  sem_ti, sem_tv, sem_z, sem_misc):
    src_hbm = ei_hbm.at[0]
    dst_hbm = ei_hbm.at[1]
    c = lax.axis_index("c")
    s = lax.axis_index("s")
    zvec = jnp.zeros((L,), jnp.float32)

    for b in range(2):
        @pl.loop(0, ROW_BLK, unroll=4)
        def _(r):
            for h in range(HALF // L):
                buf[b, r, pl.ds(h * L, L)] = zvec

    def _hz_cp(i):
        j = i * NSUB + s
        return pltpu.make_async_copy(
            buf.at[0], hacc_sh.at[pl.ds(j * ROW_BLK, ROW_BLK)], sem_misc)

    @pl.loop(0, ROW_ITERS)
    def _(i):
        @pl.when(i * NSUB + s < N_ROW_BLKS)
        def _():
            _hz_cp(i).start()

    @pl.loop(0, ROW_ITERS)
    def _(i):
        @pl.when(i * NSUB + s < N_ROW_BLKS)
        def _():
            _hz_cp(i).wait()

    @pl.when(jnp.logical_and(c == 0, s == 0))
    def _():
        @pl.loop(0, N // 5 // L)
        def _(i):
            zdeg[pl.ds(i * L, L)] = zvec

        @pl.loop(0, 5)
        def _(i):
            pltpu.make_async_copy(
                zdeg, deg_sh.at[pl.ds(i * (N // 5), N // 5)], sem_misc).start()

        @pl.loop(0, 5)
        def _(i):
            pltpu.make_async_copy(
                zdeg, deg_sh.at[pl.ds(i * (N // 5), N // 5)], sem_misc).wait()

    @pl.when(c == 0)
    def _():
        @pl.loop(0, EDGE_BLK // L)
        def _(i):
            ones[pl.ds(i * L, L)] = jnp.ones((L,), jnp.float32)

    coff = c * HALF

    def _ti_cp(i, b):
        base = (i * NSUB + s) * ROW_BLK
        return pltpu.make_async_copy(
            ti_hbm.at[pl.ds(base, ROW_BLK)], ti_vm.at[b], sem_ti.at[b])

    def _tv_cp(i, b):
        base = (i * NSUB + s) * ROW_BLK
        return pltpu.make_async_copy(
            tv_hbm.at[pl.ds(base, ROW_BLK)], tv_vm.at[b], sem_tv.at[b])

    def _bz_cp(b):
        return pltpu.make_async_copy(
            hacc_sh.at[pl.ds(s * ROW_BLK, ROW_BLK)], buf.at[b], sem_z.at[b])

    _ti_cp(0, 0).start()
    _tv_cp(0, 0).start()

    @pl.loop(0, ROW_ITERS)
    def _(i):
        @pl.when(i * NSUB + s < N_ROW_BLKS)
        def _():
            b = lax.rem(i, 2)
            _ti_cp(i, b).wait()
            _tv_cp(i, b).wait()

            @pl.when((i + 1) * NSUB + s < N_ROW_BLKS)
            def _():
                _ti_cp(i + 1, 1 - b).start()
                _tv_cp(i + 1, 1 - b).start()

            @pl.when(i >= 2)
            def _():
                _bz_cp(b).wait()

            @pl.loop(0, ROW_BLK, unroll=4)
            def _(r):
                row_ids = jnp.full((L,), r, jnp.int32)
                for h in range(K // L):
                    cols = ti_vm[b, r, pl.ds(h * L, L)] - coff
                    vals = tv_vm[b, r, pl.ds(h * L, L)]
                    mask = jnp.logical_and(cols >= 0, cols < HALF)
                    plsc.store_scatter(buf.at[b], [row_ids, cols], vals,
                                       mask=mask)

            base = (i * NSUB + s) * ROW_BLK
            pltpu.sync_copy(buf.at[b], sf_out.at[c].at[pl.ds(base, ROW_BLK)])

            @pl.when((i + 2) * NSUB + s < N_ROW_BLKS)
            def _():
                _bz_cp(b).start()

    plsc.subcore_barrier()

    start = s * BLKS_LO + jnp.minimum(s, N_EDGE_BLKS - BLKS_LO * NSUB)
    nb = BLKS_LO + jnp.where(s < N_EDGE_BLKS - BLKS_LO * NSUB, 1, 0)

    def _sidx_cp(i):
        b4 = lax.rem(i, 4)
        base = (start + i) * EDGE_BLK
        return pltpu.make_async_copy(
            src_hbm.at[pl.ds(base, EDGE_BLK)], sidx.at[b4], sem_si.at[b4])

    def _didx_cp(i):
        b4 = lax.rem(i, 4)
        base = (start + i) * EDGE_BLK
        return pltpu.make_async_copy(
            dst_hbm.at[pl.ds(base, EDGE_BLK)], didx.at[b4], sem_di.at[b4])

    def _gather_cp(i):
        b2 = lax.rem(i, 2)
        b4 = lax.rem(i, 4)
        return pltpu.make_async_copy(
            sf_out.at[c].at[sidx.at[b4]], stage.at[b2], sem_g.at[b2])

    def _scatter_cp(i):
        b2 = lax.rem(i, 2)
        b4 = lax.rem(i, 4)
        return pltpu.make_async_copy(
            stage.at[b2], hacc_sh.at[didx.at[b4]], sem_w.at[b2])

    def _deg_cp(i):
        b2 = lax.rem(i, 2)
        b4 = lax.rem(i, 4)
        return pltpu.make_async_copy(
            ones, deg_sh.at[didx.at[b4]], sem_deg.at[b2])

    def idx_start(i):
        _sidx_cp(i).start()
        _didx_cp(i).start()

    def idx_wait(i):
        _sidx_cp(i).wait()
        _didx_cp(i).wait()

    def scatter_start(i):
        _scatter_cp(i).start(add=True)

        @pl.when(c == 0)
        def _():
            _deg_cp(i).start(add=True)

    def scatter_wait(i):
        _scatter_cp(i).wait()

        @pl.when(c == 0)
        def _():
            _deg_cp(i).wait()

    idx_start(0)
    idx_start(1)
    idx_wait(0)
    _gather_cp(0).start()

    @pl.loop(1, nb)
    def _(i):
        @pl.when(i + 1 < nb)
        def _():
            idx_start(i + 1)

        idx_wait(i)

        @pl.when(i >= 2)
        def _():
            scatter_wait(i - 2)

        _gather_cp(i).start()
        _gather_cp(i - 1).wait()
        scatter_start(i - 1)

    _gather_cp(nb - 1).wait()
    scatter_start(nb - 1)
    scatter_wait(nb - 2)
    scatter_wait(nb - 1)

    plsc.subcore_barrier()

    def _out_cp(i):
        base = (i * NSUB + s) * ROW_BLK
        return pltpu.make_async_copy(
            hacc_sh.at[pl.ds(base, ROW_BLK)],
            hs_out.at[pl.ds(base, ROW_BLK), pl.ds(coff, HALF)], sem_misc)

    @pl.loop(0, ROW_ITERS)
    def _(i):
        @pl.when(i * NSUB + s < N_ROW_BLKS)
        def _():
            _out_cp(i).start()

    @pl.when(jnp.logical_and(c == 0, s == 0))
    def _():
        pltpu.sync_copy(deg_sh, deg_out)

    @pl.loop(0, ROW_ITERS)
    def _(i):
        @pl.when(i * NSUB + s < N_ROW_BLKS)
        def _():
            _out_cp(i).wait()


R_TC = 2000
_TC_GRID = N // R_TC


def _tc_body(feat_ref, h_ref, deg_ref, ws_ref, wn_ref, b_ref, out_ref):
    w = 1.0 / jnp.maximum(deg_ref[...], 1.0)
    hw = h_ref[...] * w
    dn = (((1,), (1,)), ((), ()))
    acc = lax.dot_general(feat_ref[...], ws_ref[...], dn,
                          preferred_element_type=jnp.float32)
    acc += lax.dot_general(hw, wn_ref[...], dn,
                           preferred_element_type=jnp.float32)
    out_ref[...] = acc + b_ref[...]


def _tc_combine(feat, hs, deg2d, W_self, W_neigh, b):
    return pl.pallas_call(
        _tc_body,
        grid=(_TC_GRID,),
        in_specs=[
            pl.BlockSpec((R_TC, D), lambda i: (i, 0)),
            pl.BlockSpec((R_TC, D), lambda i: (i, 0)),
            pl.BlockSpec((R_TC, 1), lambda i: (i, 0)),
            pl.BlockSpec((D, D), lambda i: (0, 0)),
            pl.BlockSpec((D, D), lambda i: (0, 0)),
            pl.BlockSpec((D,), lambda i: (0,)),
        ],
        out_specs=pl.BlockSpec((R_TC, D), lambda i: (i, 0)),
        out_shape=jax.ShapeDtypeStruct((N, D), jnp.float32),
    )(feat, hs, deg2d, W_self, W_neigh, b)


def kernel(feat, topk_values, topk_indices, edge_index, W_neigh, W_self, b_self):
    ti = topk_indices.astype(jnp.int32)
    ei = edge_index.astype(jnp.int32)
    hs, deg, _ = _sc_aggregate(topk_values.astype(jnp.float32), ti, ei)
    return _tc_combine(feat, hs, deg[:, None], W_self, W_neigh, b_self)


# speedup vs baseline: 32.8680x; 1.1189x over previous
import dataclasses
import functools

import jax
import jax.numpy as jnp
from jax import lax
from jax.experimental import pallas as pl
from jax.experimental.pallas import tpu as pltpu
from jax.experimental.pallas import tpu_sc as plsc

N = 10000
E = 320000
D = 128
K = 32
HALF = 64
L = 16
NSUB = 16
ROW_BLK = 200
N_ROW_BLKS = N // ROW_BLK
ROW_ITERS = (N_ROW_BLKS + NSUB - 1) // NSUB
EDGE_BLK = 256
N_EDGE_BLKS = E // EDGE_BLK
BLKS_LO = N_EDGE_BLKS // NSUB

_mesh = plsc.VectorSubcoreMesh(core_axis_name="c", subcore_axis_name="s")

_sc_params = pltpu.CompilerParams()
if "needs_layout_passes" in pltpu.CompilerParams.__dataclass_fields__:
    _sc_params = dataclasses.replace(_sc_params, needs_layout_passes=False)
if "use_tc_tiling_on_sc" in pltpu.CompilerParams.__dataclass_fields__:
    _sc_params = dataclasses.replace(_sc_params, use_tc_tiling_on_sc=False)


@functools.partial(
    pl.kernel,
    out_type=(
        jax.ShapeDtypeStruct((N, D), jnp.float32),
        jax.ShapeDtypeStruct((N,), jnp.float32),
        jax.ShapeDtypeStruct((2, N, HALF), jnp.float32),
    ),
    mesh=_mesh,
    compiler_params=_sc_params,
    scratch_types=[
        pltpu.VMEM_SHARED((N, HALF), jnp.float32),
        pltpu.VMEM_SHARED((N,), jnp.float32),
        pltpu.VMEM((2, ROW_BLK, HALF), jnp.float32),
        pltpu.VMEM((2, ROW_BLK, K), jnp.int32),
        pltpu.VMEM((2, ROW_BLK, K), jnp.float32),
        pltpu.VMEM((4, EDGE_BLK), jnp.int32),
        pltpu.VMEM((4, EDGE_BLK), jnp.int32),
        pltpu.VMEM((2, EDGE_BLK, HALF), jnp.float32),
        pltpu.VMEM((EDGE_BLK,), jnp.float32),
        pltpu.VMEM((N // 5,), jnp.float32),
        pltpu.SemaphoreType.DMA((4,)),
        pltpu.SemaphoreType.DMA((4,)),
        pltpu.SemaphoreType.DMA((2,)),
        pltpu.SemaphoreType.DMA((2,)),
        pltpu.SemaphoreType.DMA((2,)),
        pltpu.SemaphoreType.DMA((2,)),
        pltpu.SemaphoreType.DMA((2,)),
        pltpu.SemaphoreType.DMA((2,)),
        pltpu.SemaphoreType.DMA,
    ],
)
def _sc_aggregate(tv_hbm, ti_hbm, ei_hbm, hs_out, deg_out, sf_out,
                  hacc_sh, deg_sh,
                  buf, ti_vm, tv_vm, sidx, didx, stage, ones, zdeg,
                  sem_si, sem_di, sem_g, sem_w, sem_deg,
                  sem_ti, sem_tv, sem_z, sem_misc):
    src_hbm = ei_hbm.at[0]
    dst_hbm = ei_hbm.at[1]
    c = lax.axis_index("c")
    s = lax.axis_index("s")
    zvec = jnp.zeros((L,), jnp.float32)

    for b in range(2):
        @pl.loop(0, ROW_BLK, unroll=4)
        def _(r):
            for h in range(HALF // L):
                buf[b, r, pl.ds(h * L, L)] = zvec

    def _hz_cp(i):
        j = i * NSUB + s
        return pltpu.make_async_copy(
            buf.at[0], hacc_sh.at[pl.ds(j * ROW_BLK, ROW_BLK)], sem_misc)

    @pl.loop(0, ROW_ITERS)
    def _(i):
        @pl.when(i * NSUB + s < N_ROW_BLKS)
        def _():
            _hz_cp(i).start()

    @pl.loop(0, ROW_ITERS)
    def _(i):
        @pl.when(i * NSUB + s < N_ROW_BLKS)
        def _():
            _hz_cp(i).wait()

    @pl.when(jnp.logical_and(c == 0, s == 0))
    def _():
        @pl.loop(0, N // 5 // L)
        def _(i):
            zdeg[pl.ds(i * L, L)] = zvec

        @pl.loop(0, 5)
        def _(i):
            pltpu.make_async_copy(
                zdeg, deg_sh.at[pl.ds(i * (N // 5), N // 5)], sem_misc).start()

        @pl.loop(0, 5)
        def _(i):
            pltpu.make_async_copy(
                zdeg, deg_sh.at[pl.ds(i * (N // 5), N // 5)], sem_misc).wait()

    @pl.when(c == 0)
    def _():
        @pl.loop(0, EDGE_BLK // L)
        def _(i):
            ones[pl.ds(i * L, L)] = jnp.ones((L,), jnp.float32)

    coff = c * HALF

    def _ti_cp(i, b):
        base = (i * NSUB + s) * ROW_BLK
        return pltpu.make_async_copy(
            ti_hbm.at[pl.ds(base, ROW_BLK)], ti_vm.at[b], sem_ti.at[b])

    def _tv_cp(i, b):
        base = (i * NSUB + s) * ROW_BLK
        return pltpu.make_async_copy(
            tv_hbm.at[pl.ds(base, ROW_BLK)], tv_vm.at[b], sem_tv.at[b])

    def _bz_cp(b):
        return pltpu.make_async_copy(
            hacc_sh.at[pl.ds(s * ROW_BLK, ROW_BLK)], buf.at[b], sem_z.at[b])

    _ti_cp(0, 0).start()
    _tv_cp(0, 0).start()

    @pl.loop(0, ROW_ITERS)
    def _(i):
        @pl.when(i * NSUB + s < N_ROW_BLKS)
        def _():
            b = lax.rem(i, 2)
            _ti_cp(i, b).wait()
            _tv_cp(i, b).wait()

            @pl.when((i + 1) * NSUB + s < N_ROW_BLKS)
            def _():
                _ti_cp(i + 1, 1 - b).start()
                _tv_cp(i + 1, 1 - b).start()

            @pl.when(i >= 2)
            def _():
                _bz_cp(b).wait()

            @pl.loop(0, ROW_BLK, unroll=4)
            def _(r):
                row_ids = jnp.full((L,), r, jnp.int32)
                for h in range(K // L):
                    cols = ti_vm[b, r, pl.ds(h * L, L)] - coff
                    vals = tv_vm[b, r, pl.ds(h * L, L)]
                    mask = jnp.logical_and(cols >= 0, cols < HALF)
                    plsc.store_scatter(buf.at[b], [row_ids, cols], vals,
                                       mask=mask)

            base = (i * NSUB + s) * ROW_BLK
            pltpu.sync_copy(buf.at[b], sf_out.at[c].at[pl.ds(base, ROW_BLK)])

            @pl.when((i + 2) * NSUB + s < N_ROW_BLKS)
            def _():
                _bz_cp(b).start()

    plsc.subcore_barrier()

    start = s * BLKS_LO + jnp.minimum(s, N_EDGE_BLKS - BLKS_LO * NSUB)
    nb = BLKS_LO + jnp.where(s < N_EDGE_BLKS - BLKS_LO * NSUB, 1, 0)

    def _sidx_cp(i):
        b4 = lax.rem(i, 4)
        base = (start + i) * EDGE_BLK
        return pltpu.make_async_copy(
            src_hbm.at[pl.ds(base, EDGE_BLK)], sidx.at[b4], sem_si.at[b4])

    def _didx_cp(i):
        b4 = lax.rem(i, 4)
        base = (start + i) * EDGE_BLK
        return pltpu.make_async_copy(
            dst_hbm.at[pl.ds(base, EDGE_BLK)], didx.at[b4], sem_di.at[b4])

    def _gather_cp(i):
        b2 = lax.rem(i, 2)
        b4 = lax.rem(i, 4)
        return pltpu.make_async_copy(
            sf_out.at[c].at[sidx.at[b4]], stage.at[b2], sem_g.at[b2])

    def _scatter_cp(i):
        b2 = lax.rem(i, 2)
        b4 = lax.rem(i, 4)
        return pltpu.make_async_copy(
            stage.at[b2], hacc_sh.at[didx.at[b4]], sem_w.at[b2])

    def _deg_cp(i):
        b2 = lax.rem(i, 2)
        b4 = lax.rem(i, 4)
        return pltpu.make_async_copy(
            ones, deg_sh.at[didx.at[b4]], sem_deg.at[b2])

    def idx_start(i):
        _sidx_cp(i).start()
        _didx_cp(i).start()

    def idx_wait(i):
        _sidx_cp(i).wait()
        _didx_cp(i).wait()

    def scatter_start(i):
        _scatter_cp(i).start(add=True)

        @pl.when(c == 0)
        def _():
            _deg_cp(i).start(add=True)

    def scatter_wait(i):
        _scatter_cp(i).wait()

        @pl.when(c == 0)
        def _():
            _deg_cp(i).wait()

    idx_start(0)
    idx_start(1)
    idx_wait(0)
    _gather_cp(0).start()

    @pl.loop(1, nb)
    def _(i):
        @pl.when(i + 1 < nb)
        def _():
            idx_start(i + 1)

        idx_wait(i)

        @pl.when(i >= 2)
        def _():
            scatter_wait(i - 2)

        _gather_cp(i).start()
        _gather_cp(i - 1).wait()
        scatter_start(i - 1)

    _gather_cp(nb - 1).wait()
    scatter_start(nb - 1)
    scatter_wait(nb - 2)
    scatter_wait(nb - 1)

    plsc.subcore_barrier()

    def _out_cp(i):
        base = (i * NSUB + s) * ROW_BLK
        return pltpu.make_async_copy(
            hacc_sh.at[pl.ds(base, ROW_BLK)],
            hs_out.at[pl.ds(base, ROW_BLK), pl.ds(coff, HALF)], sem_misc)

    @pl.loop(0, ROW_ITERS)
    def _(i):
        @pl.when(i * NSUB + s < N_ROW_BLKS)
        def _():
            _out_cp(i).start()

    @pl.when(jnp.logical_and(c == 0, s == 0))
    def _():
        pltpu.sync_copy(deg_sh, deg_out)

    @pl.loop(0, ROW_ITERS)
    def _(i):
        @pl.when(i * NSUB + s < N_ROW_BLKS)
        def _():
            _out_cp(i).wait()


R_TC = 2000
_TC_GRID = N // R_TC


def _tc_body(feat_ref, h_ref, deg_ref, ws_ref, wn_ref, b_ref, out_ref):
    w = 1.0 / jnp.maximum(deg_ref[...], 1.0)
    hw = h_ref[...] * w
    dn = (((1,), (1,)), ((), ()))
    acc = lax.dot_general(feat_ref[...], ws_ref[...], dn,
                          preferred_element_type=jnp.float32)
    acc += lax.dot_general(hw, wn_ref[...], dn,
                           preferred_element_type=jnp.float32)
    out_ref[...] = acc + b_ref[...]


def _tc_combine(feat, hs, deg2d, W_self, W_neigh, b):
    return pl.pallas_call(
        _tc_body,
        grid=(_TC_GRID,),
        in_specs=[
            pl.BlockSpec((R_TC, D), lambda i: (i, 0)),
            pl.BlockSpec((R_TC, D), lambda i: (i, 0)),
            pl.BlockSpec((R_TC, 1), lambda i: (i, 0)),
            pl.BlockSpec((D, D), lambda i: (0, 0)),
            pl.BlockSpec((D, D), lambda i: (0, 0)),
            pl.BlockSpec((D,), lambda i: (0,)),
        ],
        out_specs=pl.BlockSpec((R_TC, D), lambda i: (i, 0)),
        out_shape=jax.ShapeDtypeStruct((N, D), jnp.float32),
    )(feat, hs, deg2d, W_self, W_neigh, b)


def kernel(feat, topk_values, topk_indices, edge_index, W_neigh, W_self, b_self):
    ti = topk_indices.astype(jnp.int32)
    ei = edge_index.astype(jnp.int32)
    hs, deg, _ = _sc_aggregate(topk_values.astype(jnp.float32), ti, ei)
    return _tc_combine(feat, hs, deg[:, None], W_self, W_neigh, b_self)
